# baseline (speedup 1.0000x reference)
import functools

import jax
import jax.numpy as jnp
from jax import lax
from jax.experimental import pallas as pl
from jax.experimental.pallas import tpu as pltpu
from jax.experimental.pallas import tpu_sc as plsc

NC, NS = 2, 16
NW = NC * NS
CHUNK = 128
SLAB = 1024


@functools.partial(jax.jit, static_argnums=(1, 2))
def _gather_rows(args, n_rows, d):
    idx2d, table = args
    b_per_w = n_rows // NW
    n_slabs = b_per_w // SLAB
    n_chunks = SLAB // CHUNK
    mesh = plsc.VectorSubcoreMesh(
        core_axis_name="c", subcore_axis_name="s",
        num_cores=NC, num_subcores=NS)

    @functools.partial(
        pl.kernel,
        out_type=jax.ShapeDtypeStruct((n_rows, d), jnp.float32),
        mesh=mesh,
        scratch_types=[
            pltpu.VMEM((n_chunks, CHUNK), jnp.int32),
            pltpu.VMEM((SLAB, d), jnp.float32),
            pltpu.SemaphoreType.DMA,
        ],
        compiler_params=pltpu.CompilerParams(use_tc_tiling_on_sc=False),
    )
    def k(idx_hbm, table_hbm, out_hbm, idx_v, rows_v, sem):
        wid = lax.axis_index("s") * NC + lax.axis_index("c")
        base = wid * b_per_w

        def slab_body(s, carry):
            off = pl.multiple_of(base + s * SLAB, SLAB)
            crow = pl.multiple_of(off // CHUNK, SLAB // CHUNK)
            pltpu.sync_copy(idx_hbm.at[pl.ds(crow, n_chunks)], idx_v)
            handles = [
                pltpu.async_copy(
                    table_hbm.at[idx_v.at[j]],
                    rows_v.at[pl.ds(j * CHUNK, CHUNK)],
                    sem)
                for j in range(n_chunks)
            ]
            for h in handles:
                h.wait()
            pltpu.sync_copy(rows_v, out_hbm.at[pl.ds(off, SLAB)])
            return carry

        lax.fori_loop(0, n_slabs, slab_body, 0)

    return k(idx2d, table)


def kernel(resource_ids, table):
    b, n = resource_ids.shape
    n_rows = b * n
    d = table.shape[1]
    idx2d = resource_ids.reshape(n_rows // CHUNK, CHUNK)
    out = _gather_rows((idx2d, table), n_rows, d)
    return out.reshape(b, n, d)


# speedup vs baseline: 1.0163x; 1.0163x over previous
import functools

import jax
import jax.numpy as jnp
from jax import lax
from jax.experimental import pallas as pl
from jax.experimental.pallas import tpu as pltpu
from jax.experimental.pallas import tpu_sc as plsc

NC, NS = 2, 16
NW = NC * NS
CHUNK = 128
SLAB = 512
NCH = SLAB // CHUNK


@functools.partial(jax.jit, static_argnums=(1, 2))
def _gather_rows(args, n_rows, d):
    idx2d, table = args
    b_per_w = n_rows // NW
    n_slabs = b_per_w // SLAB
    idx_rows_w = b_per_w // CHUNK
    mesh = plsc.VectorSubcoreMesh(
        core_axis_name="c", subcore_axis_name="s",
        num_cores=NC, num_subcores=NS)

    @functools.partial(
        pl.kernel,
        out_type=jax.ShapeDtypeStruct((n_rows, d), jnp.float32),
        mesh=mesh,
        scratch_types=[
            pltpu.VMEM((idx_rows_w, CHUNK), jnp.int32),
            pltpu.VMEM((SLAB, d), jnp.float32),
            pltpu.VMEM((SLAB, d), jnp.float32),
            pltpu.SemaphoreType.DMA,
            pltpu.SemaphoreType.DMA,
            pltpu.SemaphoreType.DMA,
            pltpu.SemaphoreType.DMA,
        ],
        compiler_params=pltpu.CompilerParams(use_tc_tiling_on_sc=False),
    )
    def k(idx_hbm, table_hbm, out_hbm, idx_v, rows0, rows1, g0, g1, o0, o1):
        wid = lax.axis_index("s") * NC + lax.axis_index("c")
        base = pl.multiple_of(wid * b_per_w, b_per_w)
        crow0 = pl.multiple_of(base // CHUNK, idx_rows_w)
        pltpu.sync_copy(idx_hbm.at[pl.ds(crow0, idx_rows_w)], idx_v)

        bufs = (rows0, rows1)
        gsems = (g0, g1)
        osems = (o0, o1)

        def gather_slab(s, b):
            handles = [
                pltpu.async_copy(
                    table_hbm.at[idx_v.at[s * NCH + j]],
                    bufs[b].at[pl.ds(j * CHUNK, CHUNK)],
                    gsems[b])
                for j in range(NCH)
            ]
            for h in handles:
                h.wait()

        def fire_out(s, b):
            off = pl.multiple_of(base + s * SLAB, SLAB)
            pltpu.async_copy(bufs[b], out_hbm.at[pl.ds(off, SLAB)], osems[b])

        def wait_out(b):
            pltpu.make_async_copy(
                bufs[b], out_hbm.at[pl.ds(0, SLAB)], osems[b]).wait()

        gather_slab(0, 0)
        fire_out(0, 0)
        gather_slab(1, 1)
        fire_out(1, 1)

        def pair_body(p, carry):
            for b in (0, 1):
                s = 2 * p + b
                wait_out(b)
                gather_slab(s, b)
                fire_out(s, b)
            return carry

        lax.fori_loop(1, n_slabs // 2, pair_body, 0)
        wait_out(0)
        wait_out(1)

    return k(idx2d, table)


def kernel(resource_ids, table):
    b, n = resource_ids.shape
    n_rows = b * n
    d = table.shape[1]
    idx2d = resource_ids.reshape(n_rows // CHUNK, CHUNK)
    out = _gather_rows((idx2d, table), n_rows, d)
    return out.reshape(b, n, d)
